# baseline (speedup 1.0000x reference)
import functools

import jax
import jax.numpy as jnp
from jax import lax
from jax.experimental import pallas as pl
from jax.experimental.pallas import tpu as pltpu
from jax.experimental.pallas import tpu_sc as plsc

H = 1024
L = 4
DS = [1024, 512, 256, 128]
PCOLS = 128
NEG = -1e30
M = 512
NTOK = 4096
NBR = NTOK // M
NBMAX = NTOK // M + L - 1
NS = (NBMAX + 1) * M

NC, NSUB = 2, 16
NW = NC * NSUB
RPW = NTOK // NW
CH = 64
NCH = RPW // CH


def _route_body(x_ref, wp1_ref, bp1_ref, wp2_ref, bp2_ref, lt_ref,
                lev_ref, rank_ref, cnt_ref, carry_ref):
    i = pl.program_id(0)

    @pl.when(i == 0)
    def _():
        carry_ref[...] = jnp.zeros_like(carry_ref)

    x = x_ref[...]
    hp = jnp.maximum(
        lax.dot(x.astype(jnp.bfloat16), wp1_ref[...],
                precision=lax.Precision.DEFAULT,
                preferred_element_type=jnp.float32)
        + bp1_ref[...], 0.0)
    logits = lax.dot(hp, wp2_ref[...],
                     precision=lax.Precision.DEFAULT) + bp2_ref[...]
    maxv = jnp.max(logits, axis=1, keepdims=True)
    lane = lax.broadcasted_iota(jnp.int32, logits.shape, 1)
    levels = jnp.min(jnp.where(logits == maxv, lane, PCOLS),
                     axis=1, keepdims=True)
    oh = (levels == lane).astype(jnp.bfloat16)
    csum = lax.dot(lt_ref[...], oh,
                   precision=lax.Precision.DEFAULT,
                   preferred_element_type=jnp.float32)
    rank_in = jnp.sum(oh * csum, axis=1, keepdims=True) - 1.0
    carry = carry_ref[...]
    rank = rank_in + jnp.sum(oh * carry, axis=1, keepdims=True)
    newc = carry + csum[M - 1:M, :]
    carry_ref[...] = newc
    lev_ref[...] = levels
    rank_ref[...] = rank.astype(jnp.int32)
    cnt_ref[...] = newc.astype(jnp.int32)


def _route(x, wp1, bp1, wp2, bp2, lt):
    full = lambda shape: pl.BlockSpec(shape, lambda i: (0,) * len(shape))
    return pl.pallas_call(
        _route_body,
        grid=(NBR,),
        in_specs=[
            pl.BlockSpec((M, H), lambda i: (i, 0)),
            full((H, H)), full((1, H)),
            full((H, PCOLS)), full((1, PCOLS)),
            full((M, M)),
        ],
        out_specs=[
            pl.BlockSpec((M, 1), lambda i: (i, 0)),
            pl.BlockSpec((M, 1), lambda i: (i, 0)),
            pl.BlockSpec((1, PCOLS), lambda i: (0, 0)),
        ],
        out_shape=[
            jax.ShapeDtypeStruct((NTOK, 1), jnp.int32),
            jax.ShapeDtypeStruct((NTOK, 1), jnp.int32),
            jax.ShapeDtypeStruct((1, PCOLS), jnp.int32),
        ],
        scratch_shapes=[pltpu.VMEM((1, PCOLS), jnp.float32)],
        compiler_params=pltpu.CompilerParams(
            dimension_semantics=("arbitrary",)),
    )(x, wp1, bp1, wp2, bp2, lt)


_MESH = plsc.VectorSubcoreMesh(core_axis_name="c", subcore_axis_name="s")


@functools.partial(
    pl.kernel, mesh=_MESH,
    out_type=[jax.ShapeDtypeStruct((NS, H), jnp.float32),
              jax.ShapeDtypeStruct((NTOK,), jnp.int32)],
    scratch_types=[pltpu.VMEM((CH, H), jnp.float32),
                   pltpu.VMEM((CH,), jnp.int32),
                   pltpu.VMEM((CH,), jnp.int32),
                   pltpu.VMEM((CH,), jnp.int32),
                   pltpu.VMEM((16,), jnp.int32),
                   pltpu.SemaphoreType.DMA],
)
def _sc_scatter(x_hbm, lev_hbm, rank_hbm, seg_hbm, xs_hbm, pos_hbm,
                rows_v, lev_v, rank_v, idx_v, seg_v, sem):
    wid = lax.axis_index("s") * NC + lax.axis_index("c")
    pltpu.sync_copy(seg_hbm, seg_v)
    seg_vec = seg_v[...]

    def chunk(c, carry):
        base = wid * RPW + c * CH
        pltpu.sync_copy(lev_hbm.at[pl.ds(base, CH)], lev_v)
        pltpu.sync_copy(rank_hbm.at[pl.ds(base, CH)], rank_v)
        pltpu.sync_copy(x_hbm.at[pl.ds(base, CH)], rows_v)
        for t in range(CH // 16):
            lv = lev_v[pl.ds(t * 16, 16)]
            rv = rank_v[pl.ds(t * 16, 16)]
            sv = lax.gather(
                seg_vec, lv[:, None],
                lax.GatherDimensionNumbers(
                    offset_dims=(), collapsed_slice_dims=(0,),
                    start_index_map=(0,)),
                slice_sizes=(1,),
                mode=lax.GatherScatterMode.PROMISE_IN_BOUNDS)
            idx_v[pl.ds(t * 16, 16)] = sv + rv
        pltpu.async_copy(rows_v, xs_hbm.at[idx_v], sem).wait()
        pltpu.sync_copy(idx_v, pos_hbm.at[pl.ds(base, CH)])
        return carry

    lax.fori_loop(0, NCH, chunk, 0)


def _gmm_body(lvl_pref, tok_pref, xs_ref,
              w1_0, w1_1, w1_2, w1_3, w2_0, w2_1, w2_2, w2_3,
              b1_0, b1_1, b1_2, b1_3, b2_0, b2_1, b2_2, b2_3, o_ref):
    j = pl.program_id(0)
    lvl = lvl_pref[j]
    w1s = (w1_0, w1_1, w1_2, w1_3)
    w2s = (w2_0, w2_1, w2_2, w2_3)
    b1s = (b1_0, b1_1, b1_2, b1_3)
    b2s = (b2_0, b2_1, b2_2, b2_3)
    for l in range(L):
        @pl.when(lvl == l)
        def _(l=l):
            xb = xs_ref[...].astype(jnp.bfloat16)
            h = jnp.maximum(
                lax.dot(xb, w1s[l][...], precision=lax.Precision.DEFAULT,
                        preferred_element_type=jnp.float32)
                + b1s[l][...], 0.0).astype(jnp.bfloat16)
            o_ref[...] = lax.dot(
                h, w2s[l][...], precision=lax.Precision.DEFAULT,
                preferred_element_type=jnp.float32) + b2s[l][...]


def _gmm(blk_level, blk_tok, xs, w1s, w2s, b1s, b2s):
    full = lambda shape: pl.BlockSpec(
        shape, lambda j, lv, tk: (0,) * len(shape))
    grid_spec = pltpu.PrefetchScalarGridSpec(
        num_scalar_prefetch=2,
        grid=(NBMAX,),
        in_specs=[pl.BlockSpec((M, H), lambda j, lv, tk: (tk[j], 0))]
        + [full((H, d)) for d in DS]
        + [full((d, H)) for d in DS]
        + [full((1, d)) for d in DS]
        + [full((1, H)) for _ in DS],
        out_specs=pl.BlockSpec((M, H), lambda j, lv, tk: (tk[j], 0)),
    )
    return pl.pallas_call(
        _gmm_body,
        grid_spec=grid_spec,
        out_shape=jax.ShapeDtypeStruct((NS, H), jnp.float32),
        compiler_params=pltpu.CompilerParams(
            dimension_semantics=("arbitrary",)),
    )(blk_level, blk_tok, xs, *w1s, *w2s, *b1s, *b2s)


@functools.partial(
    pl.kernel, mesh=_MESH,
    out_type=jax.ShapeDtypeStruct((NTOK, H), jnp.float32),
    scratch_types=[pltpu.VMEM((CH, H), jnp.float32),
                   pltpu.VMEM((CH,), jnp.int32),
                   pltpu.SemaphoreType.DMA],
)
def _sc_gather(os_hbm, pos_hbm, out_hbm, rows_v, idx_v, sem):
    wid = lax.axis_index("s") * NC + lax.axis_index("c")

    def chunk(c, carry):
        base = wid * RPW + c * CH
        pltpu.sync_copy(pos_hbm.at[pl.ds(base, CH)], idx_v)
        pltpu.async_copy(os_hbm.at[idx_v], rows_v, sem).wait()
        pltpu.sync_copy(rows_v, out_hbm.at[pl.ds(base, CH)])
        return carry

    lax.fori_loop(0, NCH, chunk, 0)


def _tables(cnt):
    counts = cnt[0, :L]
    nblk = (counts + M - 1) // M
    cumblk = jnp.cumsum(nblk)
    seg_start = (M * (cumblk - nblk)).astype(jnp.int32)
    total_blk = cumblk[L - 1]
    j = jnp.arange(NBMAX, dtype=jnp.int32)
    blk_level = jnp.sum(
        (j[:, None] >= cumblk[None, :]).astype(jnp.int32), axis=1)
    blk_tok = jnp.where(j < total_blk, j, NBMAX).astype(jnp.int32)
    seg16 = jnp.zeros((16,), jnp.int32).at[:L].set(seg_start)
    return blk_level, blk_tok, seg16


@jax.jit
def _run(x4, wp1, bp1, wp2, bp2, lt, w1s, w2s, b1s, b2s):
    routed = [_route(x, wp1, bp1, wp2, bp2, lt) for x in x4]
    outs = []
    for x, (lev, rank, cnt) in zip(x4, routed):
        bl, bt, seg = _tables(cnt)
        xs, pos = _sc_scatter(x, lev.reshape(NTOK), rank.reshape(NTOK), seg)
        os = _gmm(bl, bt, xs, w1s, w2s, b1s, b2s)
        outs.append(_sc_gather(os, pos))
    return outs


def kernel(keys, values,
           pw1_0, pb1_0, pw2_0, pb2_0,
           pw1_1, pb1_1, pw2_1, pb2_1,
           pw1_2, pb1_2, pw2_2, pb2_2,
           pw1_3, pb1_3, pw2_3, pb2_3,
           cw1_0, cb1_0, cw2_0, cb2_0,
           cw1_1, cb1_1, cw2_1, cb2_1,
           cw1_2, cb1_2, cw2_2, cb2_2,
           cw1_3, cb1_3, cw2_3, cb2_3):
    pw1 = [pw1_0, pw1_1, pw1_2, pw1_3]
    pb1 = [pb1_0, pb1_1, pb1_2, pb1_3]
    pw2 = [pw2_0, pw2_1, pw2_2, pw2_3]
    pb2 = [pb2_0, pb2_1, pb2_2, pb2_3]
    cw1 = [cw1_0, cw1_1, cw1_2, cw1_3]
    cb1 = [cb1_0, cb1_1, cb1_2, cb1_3]
    cw2 = [cw2_0, cw2_1, cw2_2, cw2_3]
    cb2 = [cb2_0, cb2_1, cb2_2, cb2_3]

    wp1 = jnp.concatenate(pw1, axis=1).astype(jnp.bfloat16)
    bp1 = jnp.concatenate(pb1)[None, :]
    wp2 = jnp.zeros((H, PCOLS), jnp.float32)
    for l in range(L):
        wp2 = wp2.at[l * (H // L):(l + 1) * (H // L), l].set(pw2[l][:, 0])
    bp2v = jnp.full((PCOLS,), NEG, jnp.float32)
    bp2v = bp2v.at[:L].set(jnp.concatenate(pb2))[None, :]
    lt = jnp.tril(jnp.ones((M, M), jnp.bfloat16))

    w1s = [w.astype(jnp.bfloat16) for w in cw1]
    w2s = [w.astype(jnp.bfloat16) for w in cw2]
    b1s = [b[None, :] for b in cb1]
    b2s = [b[None, :] for b in cb2]

    b, s, _ = keys.shape
    xk = keys.reshape(b * s, H)
    xv = values.reshape(b * s, H)
    half = b * s // 2
    x4 = [xk[:half], xk[half:], xv[:half], xv[half:]]
    o = _run(x4, wp1, bp1, wp2, bp2v, lt, w1s, w2s, b1s, b2s)
    out_k = jnp.concatenate([o[0], o[1]], axis=0).reshape(b, s, H)
    out_v = jnp.concatenate([o[2], o[3]], axis=0).reshape(b, s, H)
    return (out_k, out_v)


# speedup vs baseline: 1.5368x; 1.5368x over previous
import functools

import jax
import jax.numpy as jnp
from jax import lax
from jax.experimental import pallas as pl
from jax.experimental.pallas import tpu as pltpu
from jax.experimental.pallas import tpu_sc as plsc

H = 1024
L = 4
DS = [1024, 512, 256, 128]
PCOLS = 128
NEG = -1e30
M = 512
NTOK = 8192
NBR = NTOK // M
NBMAX = NTOK // M + L - 1
NS = (NBMAX + 1) * M

NC, NSUB = 2, 16
NW = NC * NSUB
RPW = NTOK // NW
CH = 64
NCH = RPW // CH


def _route_body(x_ref, wp1_ref, bp1_ref, wp2_ref, bp2_ref, lt_ref,
                lev_ref, rank_ref, cnt_ref, carry_ref):
    i = pl.program_id(0)

    @pl.when(i == 0)
    def _():
        carry_ref[...] = jnp.zeros_like(carry_ref)

    x = x_ref[...]
    hp = jnp.maximum(
        lax.dot(x.astype(jnp.bfloat16), wp1_ref[...],
                precision=lax.Precision.DEFAULT,
                preferred_element_type=jnp.float32)
        + bp1_ref[...], 0.0)
    logits = lax.dot(hp.astype(jnp.bfloat16), wp2_ref[...],
                     precision=lax.Precision.DEFAULT,
                     preferred_element_type=jnp.float32) + bp2_ref[...]
    maxv = jnp.max(logits, axis=1, keepdims=True)
    lane = lax.broadcasted_iota(jnp.int32, logits.shape, 1)
    levels = jnp.min(jnp.where(logits == maxv, lane, PCOLS),
                     axis=1, keepdims=True)
    oh = (levels == lane).astype(jnp.bfloat16)
    csum = lax.dot(lt_ref[...], oh,
                   precision=lax.Precision.DEFAULT,
                   preferred_element_type=jnp.float32)
    rank_in = jnp.sum(oh * csum, axis=1, keepdims=True) - 1.0
    carry = carry_ref[...]
    rank = rank_in + jnp.sum(oh * carry, axis=1, keepdims=True)
    newc = carry + csum[M - 1:M, :]
    carry_ref[...] = newc
    lev_ref[...] = levels
    rank_ref[...] = rank.astype(jnp.int32)
    cnt_ref[...] = newc.astype(jnp.int32)


def _route(x, wp1, bp1, wp2, bp2, lt):
    full = lambda shape: pl.BlockSpec(shape, lambda i: (0,) * len(shape))
    return pl.pallas_call(
        _route_body,
        grid=(NBR,),
        in_specs=[
            pl.BlockSpec((M, H), lambda i: (i, 0)),
            full((H, H)), full((1, H)),
            full((H, PCOLS)), full((1, PCOLS)),
            full((M, M)),
        ],
        out_specs=[
            pl.BlockSpec((M, 1), lambda i: (i, 0)),
            pl.BlockSpec((M, 1), lambda i: (i, 0)),
            pl.BlockSpec((1, PCOLS), lambda i: (0, 0)),
        ],
        out_shape=[
            jax.ShapeDtypeStruct((NTOK, 1), jnp.int32),
            jax.ShapeDtypeStruct((NTOK, 1), jnp.int32),
            jax.ShapeDtypeStruct((1, PCOLS), jnp.int32),
        ],
        scratch_shapes=[pltpu.VMEM((1, PCOLS), jnp.float32)],
        compiler_params=pltpu.CompilerParams(
            dimension_semantics=("arbitrary",)),
    )(x, wp1, bp1, wp2, bp2, lt)


_MESH = plsc.VectorSubcoreMesh(core_axis_name="c", subcore_axis_name="s")


@functools.partial(
    pl.kernel, mesh=_MESH,
    out_type=[jax.ShapeDtypeStruct((NS, H), jnp.float32),
              jax.ShapeDtypeStruct((NTOK,), jnp.int32)],
    scratch_types=[pltpu.VMEM((CH, H), jnp.float32),
                   pltpu.VMEM((CH,), jnp.int32),
                   pltpu.VMEM((CH,), jnp.int32),
                   pltpu.VMEM((CH,), jnp.int32),
                   pltpu.VMEM((16,), jnp.int32),
                   pltpu.SemaphoreType.DMA],
)
def _sc_scatter(x_hbm, lev_hbm, rank_hbm, seg_hbm, xs_hbm, pos_hbm,
                rows_v, lev_v, rank_v, idx_v, seg_v, sem):
    wid = lax.axis_index("s") * NC + lax.axis_index("c")
    pltpu.sync_copy(seg_hbm, seg_v)
    seg_vec = seg_v[...]

    def chunk(c, carry):
        base = wid * RPW + c * CH
        pltpu.sync_copy(lev_hbm.at[pl.ds(base, CH)], lev_v)
        pltpu.sync_copy(rank_hbm.at[pl.ds(base, CH)], rank_v)
        pltpu.sync_copy(x_hbm.at[pl.ds(base, CH)], rows_v)
        for t in range(CH // 16):
            lv = lev_v[pl.ds(t * 16, 16)]
            rv = rank_v[pl.ds(t * 16, 16)]
            sv = lax.gather(
                seg_vec, lv[:, None],
                lax.GatherDimensionNumbers(
                    offset_dims=(), collapsed_slice_dims=(0,),
                    start_index_map=(0,)),
                slice_sizes=(1,),
                mode=lax.GatherScatterMode.PROMISE_IN_BOUNDS)
            idx_v[pl.ds(t * 16, 16)] = sv + rv
        pltpu.async_copy(rows_v, xs_hbm.at[idx_v], sem).wait()
        pltpu.sync_copy(idx_v, pos_hbm.at[pl.ds(base, CH)])
        return carry

    lax.fori_loop(0, NCH, chunk, 0)


def _gmm_body(lvl_pref, tok_pref, xs_ref,
              w1_0, w1_1, w1_2, w1_3, w2_0, w2_1, w2_2, w2_3,
              b1_0, b1_1, b1_2, b1_3, b2_0, b2_1, b2_2, b2_3, o_ref):
    j = pl.program_id(0)
    lvl = lvl_pref[j]
    w1s = (w1_0, w1_1, w1_2, w1_3)
    w2s = (w2_0, w2_1, w2_2, w2_3)
    b1s = (b1_0, b1_1, b1_2, b1_3)
    b2s = (b2_0, b2_1, b2_2, b2_3)
    for l in range(L):
        @pl.when(lvl == l)
        def _(l=l):
            xb = xs_ref[...].astype(jnp.bfloat16)
            h = jnp.maximum(
                lax.dot(xb, w1s[l][...], precision=lax.Precision.DEFAULT,
                        preferred_element_type=jnp.float32)
                + b1s[l][...], 0.0).astype(jnp.bfloat16)
            o_ref[...] = lax.dot(
                h, w2s[l][...], precision=lax.Precision.DEFAULT,
                preferred_element_type=jnp.float32) + b2s[l][...]


def _gmm(blk_level, blk_tok, xs, w1s, w2s, b1s, b2s):
    full = lambda shape: pl.BlockSpec(
        shape, lambda j, lv, tk: (0,) * len(shape))
    grid_spec = pltpu.PrefetchScalarGridSpec(
        num_scalar_prefetch=2,
        grid=(NBMAX,),
        in_specs=[pl.BlockSpec((M, H), lambda j, lv, tk: (tk[j], 0))]
        + [full((H, d)) for d in DS]
        + [full((d, H)) for d in DS]
        + [full((1, d)) for d in DS]
        + [full((1, H)) for _ in DS],
        out_specs=pl.BlockSpec((M, H), lambda j, lv, tk: (tk[j], 0)),
    )
    return pl.pallas_call(
        _gmm_body,
        grid_spec=grid_spec,
        out_shape=jax.ShapeDtypeStruct((NS, H), jnp.float32),
        compiler_params=pltpu.CompilerParams(
            dimension_semantics=("parallel",)),
    )(blk_level, blk_tok, xs, *w1s, *w2s, *b1s, *b2s)


@functools.partial(
    pl.kernel, mesh=_MESH,
    out_type=jax.ShapeDtypeStruct((NTOK, H), jnp.float32),
    scratch_types=[pltpu.VMEM((CH, H), jnp.float32),
                   pltpu.VMEM((CH,), jnp.int32),
                   pltpu.SemaphoreType.DMA],
)
def _sc_gather(os_hbm, pos_hbm, out_hbm, rows_v, idx_v, sem):
    wid = lax.axis_index("s") * NC + lax.axis_index("c")

    def chunk(c, carry):
        base = wid * RPW + c * CH
        pltpu.sync_copy(pos_hbm.at[pl.ds(base, CH)], idx_v)
        pltpu.async_copy(os_hbm.at[idx_v], rows_v, sem).wait()
        pltpu.sync_copy(rows_v, out_hbm.at[pl.ds(base, CH)])
        return carry

    lax.fori_loop(0, NCH, chunk, 0)


def _tables(cnt):
    counts = cnt[0, :L]
    nblk = (counts + M - 1) // M
    cumblk = jnp.cumsum(nblk)
    seg_start = (M * (cumblk - nblk)).astype(jnp.int32)
    total_blk = cumblk[L - 1]
    j = jnp.arange(NBMAX, dtype=jnp.int32)
    blk_level = jnp.sum(
        (j[:, None] >= cumblk[None, :]).astype(jnp.int32), axis=1)
    blk_tok = jnp.where(j < total_blk, j, NBMAX).astype(jnp.int32)
    seg16 = jnp.zeros((16,), jnp.int32).at[:L].set(seg_start)
    return blk_level, blk_tok, seg16


@jax.jit
def _run(xk, xv, wp1, bp1, wp2, bp2, lt, w1s, w2s, b1s, b2s):
    lev_k, rank_k, cnt_k = _route(xk, wp1, bp1, wp2, bp2, lt)
    lev_v, rank_v, cnt_v = _route(xv, wp1, bp1, wp2, bp2, lt)
    bl_k, bt_k, seg_k = _tables(cnt_k)
    bl_v, bt_v, seg_v = _tables(cnt_v)
    xs_k, pos_k = _sc_scatter(xk, lev_k.reshape(NTOK), rank_k.reshape(NTOK),
                              seg_k)
    os_k = _gmm(bl_k, bt_k, xs_k, w1s, w2s, b1s, b2s)
    xs_v, pos_v = _sc_scatter(xv, lev_v.reshape(NTOK), rank_v.reshape(NTOK),
                              seg_v)
    out_k = _sc_gather(os_k, pos_k)
    os_v = _gmm(bl_v, bt_v, xs_v, w1s, w2s, b1s, b2s)
    out_v = _sc_gather(os_v, pos_v)
    return out_k, out_v


def kernel(keys, values,
           pw1_0, pb1_0, pw2_0, pb2_0,
           pw1_1, pb1_1, pw2_1, pb2_1,
           pw1_2, pb1_2, pw2_2, pb2_2,
           pw1_3, pb1_3, pw2_3, pb2_3,
           cw1_0, cb1_0, cw2_0, cb2_0,
           cw1_1, cb1_1, cw2_1, cb2_1,
           cw1_2, cb1_2, cw2_2, cb2_2,
           cw1_3, cb1_3, cw2_3, cb2_3):
    pw1 = [pw1_0, pw1_1, pw1_2, pw1_3]
    pb1 = [pb1_0, pb1_1, pb1_2, pb1_3]
    pw2 = [pw2_0, pw2_1, pw2_2, pw2_3]
    pb2 = [pb2_0, pb2_1, pb2_2, pb2_3]
    cw1 = [cw1_0, cw1_1, cw1_2, cw1_3]
    cb1 = [cb1_0, cb1_1, cb1_2, cb1_3]
    cw2 = [cw2_0, cw2_1, cw2_2, cw2_3]
    cb2 = [cb2_0, cb2_1, cb2_2, cb2_3]

    wp1 = jnp.concatenate(pw1, axis=1).astype(jnp.bfloat16)
    bp1 = jnp.concatenate(pb1)[None, :]
    wp2 = jnp.zeros((H, PCOLS), jnp.float32)
    for l in range(L):
        wp2 = wp2.at[l * (H // L):(l + 1) * (H // L), l].set(pw2[l][:, 0])
    wp2 = wp2.astype(jnp.bfloat16)
    bp2v = jnp.full((PCOLS,), NEG, jnp.float32)
    bp2v = bp2v.at[:L].set(jnp.concatenate(pb2))[None, :]
    lt = jnp.tril(jnp.ones((M, M), jnp.bfloat16))

    w1s = [w.astype(jnp.bfloat16) for w in cw1]
    w2s = [w.astype(jnp.bfloat16) for w in cw2]
    b1s = [b[None, :] for b in cb1]
    b2s = [b[None, :] for b in cb2]

    b, s, _ = keys.shape
    out_k, out_v = _run(keys.reshape(b * s, H), values.reshape(b * s, H),
                        wp1, bp1, wp2, bp2v, lt, w1s, w2s, b1s, b2s)
    return (out_k.reshape(b, s, H), out_v.reshape(b, s, H))
